# baseline (speedup 1.0000x reference)
import jax
import jax.numpy as jnp
from jax.experimental import pallas as pl
from jax.experimental.pallas import tpu as pltpu


def _dma_copy(in_ref, out_ref, sem):
    copy = pltpu.make_async_copy(in_ref, out_ref, sem)
    copy.start()
    copy.wait()


def kernel(tokens, embedding_weight):
    seq_len = tokens.shape[1]
    _, d_model = embedding_weight.shape
    out = pl.pallas_call(
        _dma_copy,
        in_specs=[pl.BlockSpec(memory_space=pl.ANY)],
        out_specs=pl.BlockSpec(memory_space=pl.ANY),
        scratch_shapes=[pltpu.SemaphoreType.DMA],
        out_shape=jax.ShapeDtypeStruct((seq_len, d_model), embedding_weight.dtype),
    )(embedding_weight)
    return out[None]


# speedup vs baseline: 1.0003x; 1.0003x over previous
import jax
import jax.numpy as jnp
from jax.experimental import pallas as pl
from jax.experimental.pallas import tpu as pltpu

_NCHUNK = 16


def _dma_copy(in_ref, out_ref, sems):
    rows = in_ref.shape[0]
    chunk = rows // _NCHUNK
    copies = []
    for i in range(_NCHUNK):
        sl = pl.ds(i * chunk, chunk)
        copies.append(pltpu.make_async_copy(in_ref.at[sl], out_ref.at[sl], sems.at[i]))
    for c in copies:
        c.start()
    for c in copies:
        c.wait()


def kernel(tokens, embedding_weight):
    seq_len = tokens.shape[1]
    _, d_model = embedding_weight.shape
    out = pl.pallas_call(
        _dma_copy,
        in_specs=[pl.BlockSpec(memory_space=pl.ANY)],
        out_specs=pl.BlockSpec(memory_space=pl.ANY),
        scratch_shapes=[pltpu.SemaphoreType.DMA((_NCHUNK,))],
        out_shape=jax.ShapeDtypeStruct((seq_len, d_model), embedding_weight.dtype),
    )(embedding_weight)
    return out[None]


# speedup vs baseline: 48.4454x; 48.4332x over previous
import jax
import jax.numpy as jnp
from jax.experimental import pallas as pl


def _copy_block(in_ref, out_ref):
    out_ref[...] = in_ref[...]


def kernel(tokens, embedding_weight):
    seq_len = tokens.shape[1]
    _, d_model = embedding_weight.shape
    block = 1024
    out = pl.pallas_call(
        _copy_block,
        grid=(seq_len // block,),
        in_specs=[pl.BlockSpec((block, d_model), lambda i: (i, 0))],
        out_specs=pl.BlockSpec((block, d_model), lambda i: (i, 0)),
        out_shape=jax.ShapeDtypeStruct((seq_len, d_model), embedding_weight.dtype),
    )(embedding_weight)
    return out[None]
